# baseline (device time: 78775 ns/iter reference)
import jax
import jax.numpy as jnp
from jax import lax
from jax.experimental import pallas as pl
from jax.experimental.pallas import tpu as pltpu

N_DEV = 4
EPS = 1e-5
CB = 256


def kernel(x, gamma):
    m, n_local = x.shape
    n_global = n_local * N_DEV
    rows = m // 128
    G = n_local // CB
    g2 = gamma.reshape(1, n_local)

    def body(x_ref, g_ref, out_ref, cache_ref, inv_ref, comm_ref,
             send_sems, recv_sems):
        i = pl.program_id(0)
        my = lax.axis_index("i")

        @pl.when(i == 0)
        def _barrier():
            barrier = pltpu.get_barrier_semaphore()
            for k in range(1, N_DEV):
                peer = lax.rem(my + k, N_DEV)
                pl.semaphore_signal(
                    barrier, inc=1,
                    device_id=(peer,), device_id_type=pl.DeviceIdType.MESH,
                )
            pl.semaphore_wait(barrier, N_DEV - 1)

        @pl.when(i < G)
        def _phase0():
            xw = x_ref[:, :]
            x3 = xw.reshape(rows, 128, CB)
            p = jnp.sum(x3 * x3, axis=2)

            @pl.when(i == 0)
            def _():
                comm_ref[0, :, :] = p

            @pl.when(i > 0)
            def _():
                comm_ref[0, :, :] = comm_ref[0, :, :] + p

            cache_ref[:, pl.ds(i * CB, CB)] = xw.astype(jnp.bfloat16)

        @pl.when(i == G - 1)
        def _exchange():
            rdmas = []
            for k in range(1, N_DEV):
                peer = lax.rem(my + k, N_DEV)
                rdma = pltpu.make_async_remote_copy(
                    src_ref=comm_ref.at[0],
                    dst_ref=comm_ref.at[N_DEV - k],
                    send_sem=send_sems.at[k - 1],
                    recv_sem=recv_sems.at[N_DEV - k],
                    device_id=(peer,),
                    device_id_type=pl.DeviceIdType.MESH,
                )
                rdma.start()
                rdmas.append(rdma)
            for rdma in rdmas:
                rdma.wait()
            total = (
                comm_ref[0, :, :] + comm_ref[1, :, :]
                + comm_ref[2, :, :] + comm_ref[3, :, :]
            )
            inv_ref[:, :] = lax.rsqrt(total / n_global + EPS)

        @pl.when(i >= G)
        def _phase1():
            j = i - G
            xc = cache_ref[:, pl.ds(j * CB, CB)].astype(jnp.float32)
            x3 = xc.reshape(rows, 128, CB)
            s3 = inv_ref[:, :].reshape(rows, 128, 1)
            gw = g_ref[:, :].reshape(1, 1, CB)
            out_ref[:, :] = (x3 * s3 * gw).reshape(m, CB)

    return pl.pallas_call(
        body,
        grid=(2 * G,),
        out_shape=jax.ShapeDtypeStruct((m, n_local), x.dtype),
        in_specs=[
            pl.BlockSpec((m, CB), lambda i: (0, jnp.minimum(i, G - 1))),
            pl.BlockSpec((1, CB), lambda i: (0, jnp.maximum(i - G, 0))),
        ],
        out_specs=pl.BlockSpec((m, CB), lambda i: (0, jnp.maximum(i - G, 0))),
        scratch_shapes=[
            pltpu.VMEM((m, n_local), jnp.bfloat16),
            pltpu.VMEM((rows, 128), jnp.float32),
            pltpu.VMEM((N_DEV, rows, 128), jnp.float32),
            pltpu.SemaphoreType.DMA((N_DEV - 1,)),
            pltpu.SemaphoreType.DMA((N_DEV,)),
        ],
        compiler_params=pltpu.CompilerParams(
            collective_id=0,
            vmem_limit_bytes=60 * 1024 * 1024,
        ),
    )(x, g2)


# device time: 72980 ns/iter; 1.0794x vs baseline; 1.0794x over previous
import jax
import jax.numpy as jnp
from jax import lax
from jax.experimental import pallas as pl
from jax.experimental.pallas import tpu as pltpu

N_DEV = 4
EPS = 1e-5
BLK = 1024


def kernel(x, gamma):
    m, n_local = x.shape
    n_global = n_local * N_DEV
    rows = m // 128
    G = m // BLK
    tb = BLK // 128
    g2 = gamma.reshape(1, n_local)

    def body(x_ref, g_ref, out_ref, cache_ref, inv_ref, comm_ref,
             send_sems, recv_sems):
        i = pl.program_id(0)
        my = lax.axis_index("i")

        @pl.when(i == 0)
        def _barrier():
            barrier = pltpu.get_barrier_semaphore()
            for k in range(1, N_DEV):
                peer = lax.rem(my + k, N_DEV)
                pl.semaphore_signal(
                    barrier, inc=1,
                    device_id=(peer,), device_id_type=pl.DeviceIdType.MESH,
                )
            pl.semaphore_wait(barrier, N_DEV - 1)

        @pl.when(i < G)
        def _phase0():
            xw = x_ref[:, :]
            x3 = xw.reshape(tb, 128, n_local)
            comm_ref[0, pl.ds(i * tb, tb), :] = jnp.sum(x3 * x3, axis=2)
            cache_ref[pl.ds(i * BLK, BLK), :] = xw.astype(jnp.bfloat16)

        @pl.when(i == G - 1)
        def _exchange():
            rdmas = []
            for k in range(1, N_DEV):
                peer = lax.rem(my + k, N_DEV)
                rdma = pltpu.make_async_remote_copy(
                    src_ref=comm_ref.at[0],
                    dst_ref=comm_ref.at[N_DEV - k],
                    send_sem=send_sems.at[k - 1],
                    recv_sem=recv_sems.at[N_DEV - k],
                    device_id=(peer,),
                    device_id_type=pl.DeviceIdType.MESH,
                )
                rdma.start()
                rdmas.append(rdma)
            for rdma in rdmas:
                rdma.wait()
            total = (
                comm_ref[0, :, :] + comm_ref[1, :, :]
                + comm_ref[2, :, :] + comm_ref[3, :, :]
            )
            inv_ref[:, :] = lax.rsqrt(total / n_global + EPS)

        @pl.when(i >= G)
        def _phase1():
            j = i - G
            xc = cache_ref[pl.ds(j * BLK, BLK), :].astype(jnp.float32)
            x3 = xc.reshape(tb, 128, n_local)
            s3 = inv_ref[pl.ds(j * tb, tb), :].reshape(tb, 128, 1)
            out_ref[:, :] = (x3 * s3 * g_ref[:, :]).reshape(BLK, n_local)

    return pl.pallas_call(
        body,
        grid=(2 * G,),
        out_shape=jax.ShapeDtypeStruct((m, n_local), x.dtype),
        in_specs=[
            pl.BlockSpec((BLK, n_local), lambda i: (jnp.minimum(i, G - 1), 0)),
            pl.BlockSpec((1, n_local), lambda i: (0, 0)),
        ],
        out_specs=pl.BlockSpec(
            (BLK, n_local), lambda i: (jnp.maximum(i - G, 0), 0)
        ),
        scratch_shapes=[
            pltpu.VMEM((m, n_local), jnp.bfloat16),
            pltpu.VMEM((rows, 128), jnp.float32),
            pltpu.VMEM((N_DEV, rows, 128), jnp.float32),
            pltpu.SemaphoreType.DMA((N_DEV - 1,)),
            pltpu.SemaphoreType.DMA((N_DEV,)),
        ],
        compiler_params=pltpu.CompilerParams(
            collective_id=0,
            vmem_limit_bytes=60 * 1024 * 1024,
        ),
    )(x, g2)


# device time: 69556 ns/iter; 1.1325x vs baseline; 1.0492x over previous
import jax
import jax.numpy as jnp
from jax import lax
from jax.experimental import pallas as pl
from jax.experimental.pallas import tpu as pltpu

N_DEV = 4
EPS = 1e-5
BLK = 1024
RING = 4
PRE = 3


def kernel(x, gamma):
    m, n_local = x.shape
    n_global = n_local * N_DEV
    G = m // BLK
    tb = BLK // 128
    g2 = gamma.reshape(1, n_local)

    def body(x_hbm, g_ref, out_ref, ring, comm_ref, dma_sems,
             send_sems, recv_sems):
        i = pl.program_id(0)
        my = lax.axis_index("i")

        def in_dma(b):
            slot = b % RING if isinstance(b, int) else lax.rem(b, RING)
            return pltpu.make_async_copy(
                x_hbm.at[pl.ds(b * BLK, BLK), :],
                ring.at[slot],
                dma_sems.at[slot],
            )

        def partial_rdma(k, b):
            return pltpu.make_async_remote_copy(
                src_ref=comm_ref.at[0, b],
                dst_ref=comm_ref.at[N_DEV - k, b],
                send_sem=send_sems.at[k - 1, b],
                recv_sem=recv_sems.at[N_DEV - k, b],
                device_id=(lax.rem(my + k, N_DEV),),
                device_id_type=pl.DeviceIdType.MESH,
            )

        @pl.when(i == 0)
        def _start():
            barrier = pltpu.get_barrier_semaphore()
            for k in range(1, N_DEV):
                peer = lax.rem(my + k, N_DEV)
                pl.semaphore_signal(
                    barrier, inc=1,
                    device_id=(peer,), device_id_type=pl.DeviceIdType.MESH,
                )
            pl.semaphore_wait(barrier, N_DEV - 1)
            for b in range(min(PRE, G)):
                in_dma(b).start()

        @pl.when((i > 0) & (i + PRE - 1 < G))
        def _prefetch():
            in_dma(i + PRE - 1).start()

        @pl.when(i < G)
        def _partial():
            in_dma(i).wait()
            x3 = ring[lax.rem(i, RING)].reshape(tb, 128, n_local)
            comm_ref[0, i] = jnp.sum(x3 * x3, axis=2)
            for k in range(1, N_DEV):
                partial_rdma(k, i).start()

        @pl.when(i > 0)
        def _scale():
            j = i - 1
            for k in range(1, N_DEV):
                partial_rdma(k, j).wait()
            total = (
                comm_ref[0, j] + comm_ref[1, j]
                + comm_ref[2, j] + comm_ref[3, j]
            )
            inv3 = lax.rsqrt(total / n_global + EPS).reshape(tb, 128, 1)
            x3 = ring[lax.rem(j, RING)].reshape(tb, 128, n_local)
            gw = g_ref[:, :].reshape(1, 1, n_local)
            out_ref[:, :] = (x3 * inv3 * gw).reshape(BLK, n_local)

    return pl.pallas_call(
        body,
        grid=(G + 1,),
        out_shape=jax.ShapeDtypeStruct((m, n_local), x.dtype),
        in_specs=[
            pl.BlockSpec(memory_space=pltpu.MemorySpace.HBM),
            pl.BlockSpec((1, n_local), lambda i: (0, 0)),
        ],
        out_specs=pl.BlockSpec(
            (BLK, n_local), lambda i: (jnp.maximum(i - 1, 0), 0)
        ),
        scratch_shapes=[
            pltpu.VMEM((RING, BLK, n_local), jnp.float32),
            pltpu.VMEM((N_DEV, G, tb, 128), jnp.float32),
            pltpu.SemaphoreType.DMA((RING,)),
            pltpu.SemaphoreType.DMA((N_DEV - 1, G)),
            pltpu.SemaphoreType.DMA((N_DEV, G)),
        ],
        compiler_params=pltpu.CompilerParams(
            collective_id=0,
            vmem_limit_bytes=60 * 1024 * 1024,
        ),
    )(x, g2)
